# Initial kernel scaffold; baseline (speedup 1.0000x reference)
import jax
import jax.numpy as jnp
from jax.experimental import pallas as pl


def kernel(alpha_bar, t_int):
    raise NotImplementedError("write your pallas kernel here")


# speedup vs baseline: 3.3008x; 3.3008x over previous
import functools

import jax
import jax.numpy as jnp
from jax import lax
from jax.experimental import pallas as pl
from jax.experimental.pallas import tpu as pltpu
from jax.experimental.pallas import tpu_sc as plsc

_NC = 2
_NS = 16
_NW = _NC * _NS


def _gather_body(table_hbm, idx_hbm, out_hbm, idx_v, out_v, sem,
                 *, b_per_w):
    wid = lax.axis_index("s") * _NC + lax.axis_index("c")
    base = wid * b_per_w
    pltpu.sync_copy(idx_hbm.at[pl.ds(base, b_per_w)], idx_v)
    pltpu.async_copy(table_hbm.at[idx_v], out_v, sem).wait()
    pltpu.sync_copy(out_v, out_hbm.at[pl.ds(base, b_per_w)])


@jax.jit
def kernel(alpha_bar, t_int):
    original_shape = t_int.shape
    flat = jnp.ravel(t_int).astype(jnp.int32)
    b = flat.shape[0]
    table = alpha_bar.astype(jnp.float32)
    b_per_w = b // _NW

    mesh = plsc.VectorSubcoreMesh(core_axis_name="c", subcore_axis_name="s")
    body = functools.partial(_gather_body, b_per_w=b_per_w)
    out = pl.kernel(
        body,
        mesh=mesh,
        out_type=jax.ShapeDtypeStruct((b,), jnp.float32),
        scratch_types=[
            pltpu.VMEM((b_per_w,), jnp.int32),
            pltpu.VMEM((b_per_w,), jnp.float32),
            pltpu.SemaphoreType.DMA,
        ],
    )(table, flat)
    return out.reshape(original_shape)


# speedup vs baseline: 4.7512x; 1.4394x over previous
import functools

import jax
import jax.numpy as jnp
from jax import lax
from jax.experimental import pallas as pl
from jax.experimental.pallas import tpu as pltpu
from jax.experimental.pallas import tpu_sc as plsc

_NC = 2
_NS = 16
_NW = _NC * _NS


def _gather_body(table_hbm, idx_hbm, out_hbm, table_sh, idx_v, out_v, sem,
                 *, b_per_w):
    sid = lax.axis_index("s")
    wid = sid * _NC + lax.axis_index("c")
    base = wid * b_per_w
    pltpu.sync_copy(idx_hbm.at[pl.ds(base, b_per_w)], idx_v)

    @pl.when(sid == 0)
    def _stage_table():
        pltpu.sync_copy(table_hbm, table_sh)

    plsc.subcore_barrier()
    pltpu.async_copy(table_sh.at[idx_v], out_v, sem).wait()
    pltpu.sync_copy(out_v, out_hbm.at[pl.ds(base, b_per_w)])


@jax.jit
def kernel(alpha_bar, t_int):
    original_shape = t_int.shape
    flat = jnp.ravel(t_int).astype(jnp.int32)
    b = flat.shape[0]
    t = alpha_bar.shape[0]
    t_pad = (t + 7) // 8 * 8
    table = jnp.pad(alpha_bar.astype(jnp.float32), (0, t_pad - t))
    b_per_w = b // _NW

    mesh = plsc.VectorSubcoreMesh(core_axis_name="c", subcore_axis_name="s")
    body = functools.partial(_gather_body, b_per_w=b_per_w)
    out = pl.kernel(
        body,
        mesh=mesh,
        out_type=jax.ShapeDtypeStruct((b,), jnp.float32),
        scratch_types=[
            pltpu.VMEM_SHARED((t_pad,), jnp.float32),
            pltpu.VMEM((b_per_w,), jnp.int32),
            pltpu.VMEM((b_per_w,), jnp.float32),
            pltpu.SemaphoreType.DMA,
        ],
    )(table, flat)
    return out.reshape(original_shape)
